# baseline (speedup 1.0000x reference)
import jax
import jax.numpy as jnp
from jax.experimental import pallas as pl

N = 4096
D = 512
BI = 128


def _body(nf_ref, e_ref, a_ref, o_ref):
    i = pl.program_id(0)
    a = a_ref[...]
    o_ref[:, :D] = nf_ref[pl.ds(i * BI, BI), :]
    o_ref[:, D:2 * D] = jax.lax.dot(
        a.astype(jnp.bfloat16), nf_ref[...].astype(jnp.bfloat16),
        preferred_element_type=jnp.float32)
    o_ref[:, 2 * D:] = jnp.sum(a * e_ref[...].T, axis=1, keepdims=True)


@jax.jit
def kernel(node_features, edge_features, adj):
    return pl.pallas_call(
        _body,
        grid=(N // BI,),
        in_specs=[
            pl.BlockSpec((N, D), lambda i: (0, 0)),
            pl.BlockSpec((N, BI), lambda i: (0, i)),
            pl.BlockSpec((BI, N), lambda i: (i, 0)),
        ],
        out_specs=pl.BlockSpec((BI, 2 * D + 1), lambda i: (i, 0)),
        out_shape=jax.ShapeDtypeStruct((N, 2 * D + 1), jnp.float32),
    )(node_features, edge_features, adj)


# speedup vs baseline: 1.1139x; 1.1139x over previous
import jax
import jax.numpy as jnp
from jax.experimental import pallas as pl

N = 4096
D = 512
BI = 256


def _body(nf_ref, e_ref, a_ref, o_ref):
    i = pl.program_id(0)
    a = a_ref[...]
    o_ref[:, :D] = nf_ref[pl.ds(i * BI, BI), :]
    o_ref[:, D:2 * D] = jax.lax.dot(
        a.astype(jnp.bfloat16), nf_ref[...].astype(jnp.bfloat16),
        preferred_element_type=jnp.float32)
    o_ref[:, 2 * D:] = jnp.sum(a * e_ref[...].T, axis=1, keepdims=True)


@jax.jit
def kernel(node_features, edge_features, adj):
    return pl.pallas_call(
        _body,
        grid=(N // BI,),
        in_specs=[
            pl.BlockSpec((N, D), lambda i: (0, 0)),
            pl.BlockSpec((N, BI), lambda i: (0, i)),
            pl.BlockSpec((BI, N), lambda i: (i, 0)),
        ],
        out_specs=pl.BlockSpec((BI, 2 * D + 1), lambda i: (i, 0)),
        out_shape=jax.ShapeDtypeStruct((N, 2 * D + 1), jnp.float32),
    )(node_features, edge_features, adj)
